# baseline (speedup 1.0000x reference)
import functools

import jax
import jax.numpy as jnp
from jax.experimental import pallas as pl
from jax.experimental.pallas import tpu as pltpu


def _se_kernel(x_ref, w1s_ref, b1_ref, w2_ref, b2_ref, o_ref):
    xs = x_ref[...]
    ssum = jnp.sum(xs, axis=1, keepdims=True, dtype=jnp.float32)
    h = jnp.dot(w1s_ref[...], ssum, preferred_element_type=jnp.float32)
    h = jnp.maximum(h + b1_ref[...], 0.0)
    g = jnp.dot(w2_ref[...], h, preferred_element_type=jnp.float32)
    g = jax.nn.sigmoid(g + b2_ref[...])
    o_ref[...] = (xs * g.astype(xs.dtype)).astype(o_ref.dtype)


def kernel(x, w1, b1, w2, b2):
    n, c, h, w = x.shape
    hw = h * w
    cr = w1.shape[0]

    x2 = x.reshape(n * c, hw)
    w1s = (w1 * (1.0 / hw)).astype(jnp.float32)
    b1c = b1.reshape(cr, 1).astype(jnp.float32)
    b2c = b2.reshape(c, 1).astype(jnp.float32)

    out = pl.pallas_call(
        _se_kernel,
        out_shape=jax.ShapeDtypeStruct((n * c, hw), x.dtype),
        grid=(n,),
        in_specs=[
            pl.BlockSpec((c, hw), lambda i: (i, 0)),
            pl.BlockSpec((cr, c), lambda i: (0, 0)),
            pl.BlockSpec((cr, 1), lambda i: (0, 0)),
            pl.BlockSpec((c, cr), lambda i: (0, 0)),
            pl.BlockSpec((c, 1), lambda i: (0, 0)),
        ],
        out_specs=pl.BlockSpec((c, hw), lambda i: (i, 0)),
        compiler_params=pltpu.CompilerParams(
            dimension_semantics=("parallel",),
            vmem_limit_bytes=56 * 1024 * 1024,
        ),
    )(x2, w1s, b1c, w2, b2c)

    return out.reshape(n, c, h, w)


# speedup vs baseline: 1.3271x; 1.3271x over previous
import jax
import jax.numpy as jnp
from jax.experimental import pallas as pl
from jax.experimental.pallas import tpu as pltpu


def _se_kernel(x_ref, w1s_ref, b1_ref, w2_ref, b2_ref, o_ref):
    xs = x_ref[0]
    s1 = jnp.sum(xs, axis=1, dtype=jnp.float32)
    m = jnp.sum(s1, axis=1, keepdims=True, dtype=jnp.float32)
    h = jnp.dot(w1s_ref[...], m, preferred_element_type=jnp.float32)
    h = jnp.maximum(h + b1_ref[...], 0.0)
    g = jnp.dot(w2_ref[...], h, preferred_element_type=jnp.float32)
    g = jax.nn.sigmoid(g + b2_ref[...])
    o_ref[0] = (xs * g[:, :, None].astype(xs.dtype)).astype(o_ref.dtype)


def kernel(x, w1, b1, w2, b2):
    n, c, h, w = x.shape
    cr = w1.shape[0]

    w1s = (w1 * (1.0 / (h * w))).astype(jnp.float32)
    b1c = b1.reshape(cr, 1).astype(jnp.float32)
    b2c = b2.reshape(c, 1).astype(jnp.float32)

    return pl.pallas_call(
        _se_kernel,
        out_shape=jax.ShapeDtypeStruct((n, c, h, w), x.dtype),
        grid=(n,),
        in_specs=[
            pl.BlockSpec((1, c, h, w), lambda i: (i, 0, 0, 0)),
            pl.BlockSpec((cr, c), lambda i: (0, 0)),
            pl.BlockSpec((cr, 1), lambda i: (0, 0)),
            pl.BlockSpec((c, cr), lambda i: (0, 0)),
            pl.BlockSpec((c, 1), lambda i: (0, 0)),
        ],
        out_specs=pl.BlockSpec((1, c, h, w), lambda i: (i, 0, 0, 0)),
        compiler_params=pltpu.CompilerParams(
            dimension_semantics=("parallel",),
            vmem_limit_bytes=64 * 1024 * 1024,
        ),
    )(x, w1s, b1c, w2, b2c)


# speedup vs baseline: 9.2956x; 7.0042x over previous
import jax
import jax.numpy as jnp
from jax.experimental import pallas as pl
from jax.experimental.pallas import tpu as pltpu


def _se_kernel(x_ref, w1s_ref, b1_ref, w2_ref, b2_ref, o_ref):
    xs = x_ref[...]
    m = jnp.sum(xs, axis=1, dtype=jnp.float32)
    h = jax.lax.dot_general(m, w1s_ref[...], (((1,), (1,)), ((), ())),
                            preferred_element_type=jnp.float32)
    h = jnp.maximum(h + b1_ref[...], 0.0)
    g = jax.lax.dot_general(h, w2_ref[...], (((1,), (1,)), ((), ())),
                            preferred_element_type=jnp.float32)
    g = jax.nn.sigmoid(g + b2_ref[...])
    o_ref[...] = (xs * g.astype(xs.dtype)[:, None, :]).astype(o_ref.dtype)


def kernel(x, w1, b1, w2, b2):
    n, c, h, w = x.shape
    hw = h * w
    cr = w1.shape[0]

    xv = jnp.transpose(x, (0, 2, 3, 1)).reshape(n, hw, c)

    w1s = (w1 * (1.0 / hw)).astype(jnp.float32)
    b1r = b1.reshape(1, cr).astype(jnp.float32)
    b2r = b2.reshape(1, c).astype(jnp.float32)

    out = pl.pallas_call(
        _se_kernel,
        out_shape=jax.ShapeDtypeStruct((n, hw, c), x.dtype),
        grid=(n,),
        in_specs=[
            pl.BlockSpec((1, hw, c), lambda i: (i, 0, 0)),
            pl.BlockSpec((cr, c), lambda i: (0, 0)),
            pl.BlockSpec((1, cr), lambda i: (0, 0)),
            pl.BlockSpec((c, cr), lambda i: (0, 0)),
            pl.BlockSpec((1, c), lambda i: (0, 0)),
        ],
        out_specs=pl.BlockSpec((1, hw, c), lambda i: (i, 0, 0)),
        compiler_params=pltpu.CompilerParams(
            dimension_semantics=("parallel",),
            vmem_limit_bytes=48 * 1024 * 1024,
        ),
    )(xv, w1s, b1r, w2, b2r)

    return out.reshape(n, h, w, c).transpose(0, 3, 1, 2)


# speedup vs baseline: 9.4185x; 1.0132x over previous
import jax
import jax.numpy as jnp
from jax.experimental import pallas as pl
from jax.experimental.pallas import tpu as pltpu


def _se_kernel(x_ref, w1s_ref, b1_ref, w2_ref, b2_ref, o_ref):
    xs = x_ref[...]
    m = jnp.sum(xs, axis=1, dtype=jnp.float32)
    h = jax.lax.dot_general(m, w1s_ref[...], (((1,), (1,)), ((), ())),
                            preferred_element_type=jnp.float32)
    h = jnp.maximum(h + b1_ref[...], 0.0)
    g = jax.lax.dot_general(h, w2_ref[...], (((1,), (1,)), ((), ())),
                            preferred_element_type=jnp.float32)
    g = jax.nn.sigmoid(g + b2_ref[...])
    o_ref[...] = (xs * g.astype(xs.dtype)[:, None, :]).astype(o_ref.dtype)


def kernel(x, w1, b1, w2, b2):
    n, c, h, w = x.shape
    hw = h * w
    cr = w1.shape[0]

    xv = jnp.transpose(x, (0, 2, 3, 1)).reshape(n, hw, c)

    w1s = (w1 * (1.0 / hw)).astype(jnp.float32)
    b1r = b1.reshape(1, cr).astype(jnp.float32)
    b2r = b2.reshape(1, c).astype(jnp.float32)

    tn = 2
    out = pl.pallas_call(
        _se_kernel,
        out_shape=jax.ShapeDtypeStruct((n, hw, c), x.dtype),
        grid=(n // tn,),
        in_specs=[
            pl.BlockSpec((tn, hw, c), lambda i: (i, 0, 0)),
            pl.BlockSpec((cr, c), lambda i: (0, 0)),
            pl.BlockSpec((1, cr), lambda i: (0, 0)),
            pl.BlockSpec((c, cr), lambda i: (0, 0)),
            pl.BlockSpec((1, c), lambda i: (0, 0)),
        ],
        out_specs=pl.BlockSpec((tn, hw, c), lambda i: (i, 0, 0)),
        compiler_params=pltpu.CompilerParams(
            dimension_semantics=("parallel",),
            vmem_limit_bytes=56 * 1024 * 1024,
        ),
    )(xv, w1s, b1r, w2, b2r)

    return out.reshape(n, h, w, c).transpose(0, 3, 1, 2)
